# baseline (speedup 1.0000x reference)
import jax
import jax.numpy as jnp
from jax.experimental import pallas as pl
from jax.experimental.pallas import tpu as pltpu

P = 8
T = 8
NB = 32
H = 64
EPS = 1e-5
L = 128
NG = 4

BLK1 = 8192
BLK3 = 2048
HI = jax.lax.Precision.HIGHEST


def _stats_x_kernel(x_ref, s_ref, ss_ref):
    xA = x_ref[:, 0:P]
    s_ref[0, 0:1, :] = jnp.sum(xA, axis=0, keepdims=True)
    ss_ref[0, 0:1, :] = jnp.sum(xA * xA, axis=0, keepdims=True)


def _h1(x_ref, m0_ref, s0_ref, g0_ref, b0_ref, W1t_ref):
    xA = x_ref[:, 0:P]
    h0 = (xA - m0_ref[...]) / s0_ref[...] * g0_ref[...] + b0_ref[...]
    return jnp.dot(h0, W1t_ref[...], preferred_element_type=jnp.float32)


def _stats_h1_kernel(x_ref, m0_ref, s0_ref, g0_ref, b0_ref, W1t_ref,
                     s_ref, ss_ref):
    h1 = _h1(x_ref, m0_ref, s0_ref, g0_ref, b0_ref, W1t_ref)
    s_ref[0, 0:1, :] = jnp.sum(h1, axis=0, keepdims=True)
    ss_ref[0, 0:1, :] = jnp.sum(h1 * h1, axis=0, keepdims=True)


def _stats_h2_kernel(x_ref, m0_ref, s0_ref, g0_ref, b0_ref, W1t_ref,
                     m1_ref, s1_ref, g1_ref, b1_ref, W2t_ref,
                     s_ref, ss_ref):
    h1 = _h1(x_ref, m0_ref, s0_ref, g0_ref, b0_ref, W1t_ref)
    r1 = jnp.maximum(
        (h1 - m1_ref[...]) / s1_ref[...] * g1_ref[...] + b1_ref[...], 0.0)
    h2 = jnp.dot(r1, W2t_ref[...], preferred_element_type=jnp.float32)
    s_ref[0, 0:1, :] = jnp.sum(h2, axis=0, keepdims=True)
    ss_ref[0, 0:1, :] = jnp.sum(h2 * h2, axis=0, keepdims=True)


def _rnd(a):
    return a.astype(jnp.bfloat16).astype(jnp.float32)


def _final_kernel(x_ref, m0_ref, s0_ref, g0_ref, b0_ref, W1t_ref,
                  m1_ref, s1_ref, g1_ref, b1_ref, W2t_ref,
                  m2_ref, s2_ref, g2_ref, b2_ref,
                  Wvlo_ref, Wvhi_ref, Ww_ref, bvlo_ref, bvhi_ref, bw_ref,
                  o_ref):
    f32 = jnp.float32
    x = x_ref[...]
    h1 = _h1(x_ref, m0_ref, s0_ref, g0_ref, b0_ref, W1t_ref)
    r1 = jnp.maximum(
        (h1 - m1_ref[...]) / s1_ref[...] * g1_ref[...] + b1_ref[...], 0.0)
    h2 = jnp.dot(r1, W2t_ref[...], preferred_element_type=jnp.float32)
    r2 = jnp.maximum(
        (h2 - m2_ref[...]) / s2_ref[...] * g2_ref[...] + b2_ref[...], 0.0)
    blk = x.shape[0]

    def dot(a, b):
        return jnp.dot(a, b, preferred_element_type=f32)

    def hdot(a1, a2, b):
        return dot(a1, b) + dot(a2, b)

    r2d = jax.lax.broadcasted_iota(jnp.int32, (L, L), 0)
    c2d = jax.lax.broadcasted_iota(jnp.int32, (L, L), 1)
    tri = ((r2d <= c2d) & ((r2d // NB) == (c2d // NB))).astype(f32)
    rg = jax.lax.broadcasted_iota(jnp.int32, (L, T), 0)
    cg = jax.lax.broadcasted_iota(jnp.int32, (L, T), 1)
    lane1 = jax.lax.broadcasted_iota(jnp.int32, (1, L), 1)
    vs = ((lane1 % NB) != 0).astype(f32)
    ve = ((lane1 % NB) != NB - 1).astype(f32)
    z1 = jnp.zeros((blk, 1), f32)

    o_ref[:, 0:P] = x[:, 0:P]
    xB8 = x[:, P:P + T]
    xB8 = jnp.where(xB8 > 1.0 - 1e-6, 1.0 - 1e-6, xB8)

    Wn8 = jnp.zeros((blk, T), f32)
    Cl8 = jnp.zeros((blk, T), f32)
    Wd8 = jnp.zeros((blk, T), f32)
    lo8 = jnp.zeros((blk, T), f32)
    sh8 = jnp.zeros((blk, T), f32)
    Vd8 = jnp.zeros((blk, T), f32)
    Vh8 = jnp.zeros((blk, T), f32)
    for g in range(T // NG):
        s = slice(g, g + 1)
        G8 = ((rg // NB) == (cg - NG * g)).astype(f32)
        L31 = (((rg % NB) == NB - 1) & ((rg // NB) == (cg - NG * g))).astype(f32)
        rE = jax.lax.broadcasted_iota(jnp.int32, (T, L), 0)
        cE = jax.lax.broadcasted_iota(jnp.int32, (T, L), 1)
        E = ((cE // NB) == (rE - NG * g)).astype(f32)
        Wr = jnp.exp(dot(r2, Ww_ref[g]) + bw_ref[s, :])
        Vlo = jnp.exp(dot(r2, Wvlo_ref[g]) + bvlo_ref[s, :])
        Vhi = jnp.exp(dot(r2, Wvhi_ref[g]) + bvhi_ref[s, :])
        Wr1 = _rnd(Wr); Wr2 = Wr - Wr1
        Wcs = hdot(Wr1, Wr2, tri)
        mid = ((Vlo + Vhi) * 0.5) * Wr
        mid1 = _rnd(mid); mid2 = mid - mid1
        Craw = hdot(mid1, mid2, tri)
        Wcs1 = _rnd(Wcs); Wcs2 = Wcs - Wcs1
        Craw1 = _rnd(Craw); Craw2 = Craw - Craw1
        Wng = hdot(Wcs1, Wcs2, L31)
        Clg = hdot(Craw1, Craw2, L31)
        Wn8 = Wn8 + Wng
        Cl8 = Cl8 + Clg

        xbw = xB8 * Wng
        xbw1 = _rnd(xbw); xbw2 = xbw - xbw1
        xBW = hdot(xbw1, xbw2, E)
        cf = (Wcs > xBW).astype(f32)
        cfs = jnp.concatenate([z1, cf[:, :L - 1]], axis=1) * vs
        mf = cf - cfs
        mfl = jnp.concatenate([mf[:, 1:], z1], axis=1) * ve

        Vlo1 = _rnd(Vlo); Vlo2 = Vlo - Vlo1
        Vhi1 = _rnd(Vhi); Vhi2 = Vhi - Vhi1
        Wd8 = Wd8 + hdot(mf * Wr1, mf * Wr2, G8)
        lo8 = lo8 + hdot(mfl * Wcs1, mfl * Wcs2, G8)
        sh8 = sh8 + hdot(mfl * Craw1, mfl * Craw2, G8)
        Vd8 = Vd8 + hdot(mf * Vlo1, mf * Vlo2, G8)
        Vh8 = Vh8 + hdot(mf * Vhi1, mf * Vhi2, G8)

    Wd = Wd8 / Wn8
    lo = lo8 / Wn8
    sh = sh8 / Cl8
    Vtot = Cl8 / Wn8
    Vd = Vd8 / Vtot
    Vd1 = Vh8 / Vtot
    alpha = (xB8 - lo) / Wd
    dV = Vd1 - Vd
    o_ref[:, P:P + T] = \
        (0.5 * alpha * alpha) * (dV * Wd) + alpha * (Vd * Wd) + sh
    l = Vd + alpha * dV
    o_ref[:, P + T:P + T + 1] = x[:, P + T:P + T + 1] * \
        ((l[:, 0:1] * l[:, 1:2]) * (l[:, 2:3] * l[:, 3:4])) * \
        ((l[:, 4:5] * l[:, 5:6]) * (l[:, 6:7] * l[:, 7:8]))


def _colspec(n):
    return pl.BlockSpec((1, n), lambda i: (0, 0))


def _accspec(n):
    return pl.BlockSpec((1, 1, n), lambda i: (i, 0, 0))


def kernel(x, bn0_g, bn0_b, W1, bn1_g, bn1_b, W2, bn2_g, bn2_b, W3, b3):
    B, C = x.shape
    f32 = jnp.float32
    n1 = B // BLK1
    xspec1 = pl.BlockSpec((BLK1, C), lambda i: (i, 0))
    accshape = lambda n: jax.ShapeDtypeStruct((n1, 1, n), f32)

    s, ss = pl.pallas_call(
        _stats_x_kernel,
        grid=(n1,),
        in_specs=[xspec1],
        out_specs=[_accspec(P), _accspec(P)],
        out_shape=[accshape(P), accshape(P)],
    )(x)
    m0 = jnp.sum(s, axis=0) / B
    v0 = jnp.sum(ss, axis=0) / B - m0 * m0
    s0 = jnp.sqrt(v0 + EPS)
    g0 = bn0_g[None, :]
    b0 = bn0_b[None, :]
    W1t = W1.T

    s, ss = pl.pallas_call(
        _stats_h1_kernel,
        grid=(n1,),
        in_specs=[xspec1, _colspec(P), _colspec(P), _colspec(P), _colspec(P),
                  pl.BlockSpec((P, H), lambda i: (0, 0))],
        out_specs=[_accspec(H), _accspec(H)],
        out_shape=[accshape(H), accshape(H)],
    )(x, m0, s0, g0, b0, W1t)
    m1 = jnp.sum(s, axis=0) / B
    v1 = jnp.sum(ss, axis=0) / B - m1 * m1
    s1 = jnp.sqrt(v1 + EPS)
    g1 = bn1_g[None, :]
    b1 = bn1_b[None, :]
    W2t = W2.T

    s, ss = pl.pallas_call(
        _stats_h2_kernel,
        grid=(n1,),
        in_specs=[xspec1, _colspec(P), _colspec(P), _colspec(P), _colspec(P),
                  pl.BlockSpec((P, H), lambda i: (0, 0)),
                  _colspec(H), _colspec(H), _colspec(H), _colspec(H),
                  pl.BlockSpec((H, H), lambda i: (0, 0))],
        out_specs=[_accspec(H), _accspec(H)],
        out_shape=[accshape(H), accshape(H)],
    )(x, m0, s0, g0, b0, W1t, m1, s1, g1, b1, W2t)
    m2 = jnp.sum(s, axis=0) / B
    v2 = jnp.sum(ss, axis=0) / B - m2 * m2
    s2 = jnp.sqrt(v2 + EPS)
    g2 = bn2_g[None, :]
    b2 = bn2_b[None, :]

    NV = NB + 1
    W3r = W3.reshape(T, 2 * NB + 1, H)
    b3r = b3.reshape(T, 2 * NB + 1)
    ngrp = T // NG

    def pack(sl):
        w = W3r[:, sl, :].transpose(0, 2, 1)
        w = w.reshape(ngrp, NG, H, NB).transpose(0, 2, 1, 3)
        return w.reshape(ngrp, H, NG * NB)

    def packb(sl):
        bb = b3r[:, sl]
        return bb.reshape(ngrp, NG * NB)

    Wvlo = pack(slice(0, NB))
    Wvhi = pack(slice(1, NV))
    Ww = pack(slice(NV, NV + NB))
    bvlo = packb(slice(0, NB))
    bvhi = packb(slice(1, NV))
    bw = packb(slice(NV, NV + NB))

    n3 = B // BLK3
    wspec = pl.BlockSpec((ngrp, H, NG * NB), lambda i: (0, 0, 0))
    bspec = pl.BlockSpec((ngrp, NG * NB), lambda i: (0, 0))
    out = pl.pallas_call(
        _final_kernel,
        grid=(n3,),
        in_specs=[pl.BlockSpec((BLK3, C), lambda i: (i, 0)),
                  _colspec(P), _colspec(P), _colspec(P), _colspec(P),
                  pl.BlockSpec((P, H), lambda i: (0, 0)),
                  _colspec(H), _colspec(H), _colspec(H), _colspec(H),
                  pl.BlockSpec((H, H), lambda i: (0, 0)),
                  _colspec(H), _colspec(H), _colspec(H), _colspec(H),
                  wspec, wspec, wspec, bspec, bspec, bspec],
        out_specs=pl.BlockSpec((BLK3, C), lambda i: (i, 0)),
        out_shape=jax.ShapeDtypeStruct((B, C), f32),
        compiler_params=pltpu.CompilerParams(
            dimension_semantics=("parallel",)),
    )(x, m0, s0, g0, b0, W1t, m1, s1, g1, b1, W2t, m2, s2, g2, b2,
      Wvlo, Wvhi, Ww, bvlo, bvhi, bw)
    return out


# speedup vs baseline: 1.0818x; 1.0818x over previous
import jax
import jax.numpy as jnp
from jax.experimental import pallas as pl
from jax.experimental.pallas import tpu as pltpu

P = 8
T = 8
NB = 32
H = 64
EPS = 1e-5
L = 128
NG = 4

BLK1 = 8192
BLK3 = 4096
HI = jax.lax.Precision.HIGHEST


def _stats_x_kernel(x_ref, s_ref, ss_ref):
    xA = x_ref[:, 0:P]
    s_ref[0, 0:1, :] = jnp.sum(xA, axis=0, keepdims=True)
    ss_ref[0, 0:1, :] = jnp.sum(xA * xA, axis=0, keepdims=True)


def _h1(x_ref, m0_ref, s0_ref, g0_ref, b0_ref, W1t_ref):
    xA = x_ref[:, 0:P]
    h0 = (xA - m0_ref[...]) / s0_ref[...] * g0_ref[...] + b0_ref[...]
    return jnp.dot(h0, W1t_ref[...], preferred_element_type=jnp.float32)


def _stats_h1_kernel(x_ref, m0_ref, s0_ref, g0_ref, b0_ref, W1t_ref,
                     s_ref, ss_ref):
    h1 = _h1(x_ref, m0_ref, s0_ref, g0_ref, b0_ref, W1t_ref)
    s_ref[0, 0:1, :] = jnp.sum(h1, axis=0, keepdims=True)
    ss_ref[0, 0:1, :] = jnp.sum(h1 * h1, axis=0, keepdims=True)


def _stats_h2_kernel(x_ref, m0_ref, s0_ref, g0_ref, b0_ref, W1t_ref,
                     m1_ref, s1_ref, g1_ref, b1_ref, W2t_ref,
                     s_ref, ss_ref):
    h1 = _h1(x_ref, m0_ref, s0_ref, g0_ref, b0_ref, W1t_ref)
    r1 = jnp.maximum(
        (h1 - m1_ref[...]) / s1_ref[...] * g1_ref[...] + b1_ref[...], 0.0)
    h2 = jnp.dot(r1, W2t_ref[...], preferred_element_type=jnp.float32)
    s_ref[0, 0:1, :] = jnp.sum(h2, axis=0, keepdims=True)
    ss_ref[0, 0:1, :] = jnp.sum(h2 * h2, axis=0, keepdims=True)


def _rnd(a):
    return a.astype(jnp.bfloat16).astype(jnp.float32)


def _final_kernel(x_ref, m0_ref, s0_ref, g0_ref, b0_ref, W1t_ref,
                  m1_ref, s1_ref, g1_ref, b1_ref, W2t_ref,
                  m2_ref, s2_ref, g2_ref, b2_ref,
                  Wvlo_ref, Wvhi_ref, Ww_ref, bvlo_ref, bvhi_ref, bw_ref,
                  o_ref):
    f32 = jnp.float32
    x = x_ref[...]
    h1 = _h1(x_ref, m0_ref, s0_ref, g0_ref, b0_ref, W1t_ref)
    r1 = jnp.maximum(
        (h1 - m1_ref[...]) / s1_ref[...] * g1_ref[...] + b1_ref[...], 0.0)
    h2 = jnp.dot(r1, W2t_ref[...], preferred_element_type=jnp.float32)
    r2 = jnp.maximum(
        (h2 - m2_ref[...]) / s2_ref[...] * g2_ref[...] + b2_ref[...], 0.0)
    blk = x.shape[0]

    def dot(a, b):
        return jnp.dot(a, b, preferred_element_type=f32)

    def hdot(a1, a2, b):
        return dot(a1, b) + dot(a2, b)

    r2d = jax.lax.broadcasted_iota(jnp.int32, (L, L), 0)
    c2d = jax.lax.broadcasted_iota(jnp.int32, (L, L), 1)
    tri = ((r2d <= c2d) & ((r2d // NB) == (c2d // NB))).astype(f32)
    rg = jax.lax.broadcasted_iota(jnp.int32, (L, T), 0)
    cg = jax.lax.broadcasted_iota(jnp.int32, (L, T), 1)
    lane1 = jax.lax.broadcasted_iota(jnp.int32, (1, L), 1)
    vs = ((lane1 % NB) != 0).astype(f32)
    ve = ((lane1 % NB) != NB - 1).astype(f32)
    z1 = jnp.zeros((blk, 1), f32)

    o_ref[:, 0:P] = x[:, 0:P]
    xB8 = x[:, P:P + T]
    xB8 = jnp.where(xB8 > 1.0 - 1e-6, 1.0 - 1e-6, xB8)

    Wn8 = jnp.zeros((blk, T), f32)
    Cl8 = jnp.zeros((blk, T), f32)
    Wd8 = jnp.zeros((blk, T), f32)
    lo8 = jnp.zeros((blk, T), f32)
    sh8 = jnp.zeros((blk, T), f32)
    Vd8 = jnp.zeros((blk, T), f32)
    Vh8 = jnp.zeros((blk, T), f32)
    is31 = ((lane1 % NB) == NB - 1).astype(jnp.float32)
    for g in range(T // NG):
        s = slice(g, g + 1)
        G8 = ((rg // NB) == (cg - NG * g)).astype(f32)
        rE = jax.lax.broadcasted_iota(jnp.int32, (T, L), 0)
        cE = jax.lax.broadcasted_iota(jnp.int32, (T, L), 1)
        E = ((cE // NB) == (rE - NG * g)).astype(f32)
        Wr = jnp.exp(dot(r2, Ww_ref[g]) + bw_ref[s, :])
        Vlo = jnp.exp(dot(r2, Wvlo_ref[g]) + bvlo_ref[s, :])
        Vhi = jnp.exp(dot(r2, Wvhi_ref[g]) + bvhi_ref[s, :])
        Wr1 = _rnd(Wr); Wr2 = Wr - Wr1
        Wcs = hdot(Wr1, Wr2, tri)
        mid = ((Vlo + Vhi) * 0.5) * Wr
        mid1 = _rnd(mid); mid2 = mid - mid1
        Wng = hdot(Wr1, Wr2, G8)
        Wn8 = Wn8 + Wng
        Cl8 = Cl8 + hdot(mid1, mid2, G8)

        xbw = xB8 * Wng
        xbw1 = _rnd(xbw); xbw2 = xbw - xbw1
        xBW = hdot(xbw1, xbw2, E)
        cf = jnp.maximum((Wcs > xBW).astype(f32), is31)
        cfs = jnp.concatenate([z1, cf[:, :L - 1]], axis=1) * vs
        mf = cf - cfs
        ncf = 1.0 - cf

        Vlo1 = _rnd(Vlo); Vlo2 = Vlo - Vlo1
        Vhi1 = _rnd(Vhi); Vhi2 = Vhi - Vhi1
        Wd8 = Wd8 + hdot(mf * Wr1, mf * Wr2, G8)
        lo8 = lo8 + hdot(ncf * Wr1, ncf * Wr2, G8)
        sh8 = sh8 + hdot(ncf * mid1, ncf * mid2, G8)
        Vd8 = Vd8 + hdot(mf * Vlo1, mf * Vlo2, G8)
        Vh8 = Vh8 + hdot(mf * Vhi1, mf * Vhi2, G8)

    Wd = Wd8 / Wn8
    lo = lo8 / Wn8
    sh = sh8 / Cl8
    Vtot = Cl8 / Wn8
    Vd = Vd8 / Vtot
    Vd1 = Vh8 / Vtot
    alpha = (xB8 - lo) / Wd
    dV = Vd1 - Vd
    o_ref[:, P:P + T] = \
        (0.5 * alpha * alpha) * (dV * Wd) + alpha * (Vd * Wd) + sh
    l = Vd + alpha * dV
    o_ref[:, P + T:P + T + 1] = x[:, P + T:P + T + 1] * \
        ((l[:, 0:1] * l[:, 1:2]) * (l[:, 2:3] * l[:, 3:4])) * \
        ((l[:, 4:5] * l[:, 5:6]) * (l[:, 6:7] * l[:, 7:8]))


def _colspec(n):
    return pl.BlockSpec((1, n), lambda i: (0, 0))


def _accspec(n):
    return pl.BlockSpec((1, 1, n), lambda i: (i, 0, 0))


def kernel(x, bn0_g, bn0_b, W1, bn1_g, bn1_b, W2, bn2_g, bn2_b, W3, b3):
    B, C = x.shape
    f32 = jnp.float32
    n1 = B // BLK1
    xspec1 = pl.BlockSpec((BLK1, C), lambda i: (i, 0))
    accshape = lambda n: jax.ShapeDtypeStruct((n1, 1, n), f32)

    s, ss = pl.pallas_call(
        _stats_x_kernel,
        grid=(n1,),
        in_specs=[xspec1],
        out_specs=[_accspec(P), _accspec(P)],
        out_shape=[accshape(P), accshape(P)],
    )(x)
    m0 = jnp.sum(s, axis=0) / B
    v0 = jnp.sum(ss, axis=0) / B - m0 * m0
    s0 = jnp.sqrt(v0 + EPS)
    g0 = bn0_g[None, :]
    b0 = bn0_b[None, :]
    W1t = W1.T

    s, ss = pl.pallas_call(
        _stats_h1_kernel,
        grid=(n1,),
        in_specs=[xspec1, _colspec(P), _colspec(P), _colspec(P), _colspec(P),
                  pl.BlockSpec((P, H), lambda i: (0, 0))],
        out_specs=[_accspec(H), _accspec(H)],
        out_shape=[accshape(H), accshape(H)],
    )(x, m0, s0, g0, b0, W1t)
    m1 = jnp.sum(s, axis=0) / B
    v1 = jnp.sum(ss, axis=0) / B - m1 * m1
    s1 = jnp.sqrt(v1 + EPS)
    g1 = bn1_g[None, :]
    b1 = bn1_b[None, :]
    W2t = W2.T

    s, ss = pl.pallas_call(
        _stats_h2_kernel,
        grid=(n1,),
        in_specs=[xspec1, _colspec(P), _colspec(P), _colspec(P), _colspec(P),
                  pl.BlockSpec((P, H), lambda i: (0, 0)),
                  _colspec(H), _colspec(H), _colspec(H), _colspec(H),
                  pl.BlockSpec((H, H), lambda i: (0, 0))],
        out_specs=[_accspec(H), _accspec(H)],
        out_shape=[accshape(H), accshape(H)],
    )(x, m0, s0, g0, b0, W1t, m1, s1, g1, b1, W2t)
    m2 = jnp.sum(s, axis=0) / B
    v2 = jnp.sum(ss, axis=0) / B - m2 * m2
    s2 = jnp.sqrt(v2 + EPS)
    g2 = bn2_g[None, :]
    b2 = bn2_b[None, :]

    NV = NB + 1
    W3r = W3.reshape(T, 2 * NB + 1, H)
    b3r = b3.reshape(T, 2 * NB + 1)
    ngrp = T // NG

    def pack(sl):
        w = W3r[:, sl, :].transpose(0, 2, 1)
        w = w.reshape(ngrp, NG, H, NB).transpose(0, 2, 1, 3)
        return w.reshape(ngrp, H, NG * NB)

    def packb(sl):
        bb = b3r[:, sl]
        return bb.reshape(ngrp, NG * NB)

    Wvlo = pack(slice(0, NB))
    Wvhi = pack(slice(1, NV))
    Ww = pack(slice(NV, NV + NB))
    bvlo = packb(slice(0, NB))
    bvhi = packb(slice(1, NV))
    bw = packb(slice(NV, NV + NB))

    n3 = B // BLK3
    wspec = pl.BlockSpec((ngrp, H, NG * NB), lambda i: (0, 0, 0))
    bspec = pl.BlockSpec((ngrp, NG * NB), lambda i: (0, 0))
    out = pl.pallas_call(
        _final_kernel,
        grid=(n3,),
        in_specs=[pl.BlockSpec((BLK3, C), lambda i: (i, 0)),
                  _colspec(P), _colspec(P), _colspec(P), _colspec(P),
                  pl.BlockSpec((P, H), lambda i: (0, 0)),
                  _colspec(H), _colspec(H), _colspec(H), _colspec(H),
                  pl.BlockSpec((H, H), lambda i: (0, 0)),
                  _colspec(H), _colspec(H), _colspec(H), _colspec(H),
                  wspec, wspec, wspec, bspec, bspec, bspec],
        out_specs=pl.BlockSpec((BLK3, C), lambda i: (i, 0)),
        out_shape=jax.ShapeDtypeStruct((B, C), f32),
        compiler_params=pltpu.CompilerParams(
            dimension_semantics=("parallel",)),
    )(x, m0, s0, g0, b0, W1t, m1, s1, g1, b1, W2t, m2, s2, g2, b2,
      Wvlo, Wvhi, Ww, bvlo, bvhi, bw)
    return out


# speedup vs baseline: 1.0856x; 1.0035x over previous
import jax
import jax.numpy as jnp
from jax.experimental import pallas as pl
from jax.experimental.pallas import tpu as pltpu

P = 8
T = 8
NB = 32
H = 64
EPS = 1e-5
L = 128
NG = 4

BLK1 = 8192
BLK3 = 4096
HI = jax.lax.Precision.HIGHEST


def _stats_x_kernel(x_ref, s_ref, ss_ref):
    xA = x_ref[:, 0:P]
    s_ref[0, 0:1, :] = jnp.sum(xA, axis=0, keepdims=True)
    ss_ref[0, 0:1, :] = jnp.sum(xA * xA, axis=0, keepdims=True)


def _h1(x_ref, m0_ref, s0_ref, g0_ref, b0_ref, W1t_ref):
    xA = x_ref[:, 0:P]
    h0 = (xA - m0_ref[...]) / s0_ref[...] * g0_ref[...] + b0_ref[...]
    return jnp.dot(h0, W1t_ref[...], preferred_element_type=jnp.float32)


def _stats_h1_kernel(x_ref, m0_ref, s0_ref, g0_ref, b0_ref, W1t_ref,
                     s_ref, ss_ref):
    h1 = _h1(x_ref, m0_ref, s0_ref, g0_ref, b0_ref, W1t_ref)
    s_ref[0, 0:1, :] = jnp.sum(h1, axis=0, keepdims=True)
    ss_ref[0, 0:1, :] = jnp.sum(h1 * h1, axis=0, keepdims=True)


def _stats_h2_kernel(x_ref, m0_ref, s0_ref, g0_ref, b0_ref, W1t_ref,
                     m1_ref, s1_ref, g1_ref, b1_ref, W2t_ref,
                     s_ref, ss_ref):
    h1 = _h1(x_ref, m0_ref, s0_ref, g0_ref, b0_ref, W1t_ref)
    r1 = jnp.maximum(
        (h1 - m1_ref[...]) / s1_ref[...] * g1_ref[...] + b1_ref[...], 0.0)
    h2 = jnp.dot(r1, W2t_ref[...], preferred_element_type=jnp.float32)
    s_ref[0, 0:1, :] = jnp.sum(h2, axis=0, keepdims=True)
    ss_ref[0, 0:1, :] = jnp.sum(h2 * h2, axis=0, keepdims=True)


def _rnd(a):
    return a.astype(jnp.bfloat16).astype(jnp.float32)


def _final_kernel(x_ref, m0_ref, s0_ref, g0_ref, b0_ref, W1t_ref,
                  m1_ref, s1_ref, g1_ref, b1_ref, W2t_ref,
                  m2_ref, s2_ref, g2_ref, b2_ref,
                  Wvlo_ref, Wvhi_ref, Ww_ref, bvlo_ref, bvhi_ref, bw_ref,
                  o_ref):
    f32 = jnp.float32
    x = x_ref[...]
    h1 = _h1(x_ref, m0_ref, s0_ref, g0_ref, b0_ref, W1t_ref)
    r1 = jnp.maximum(
        (h1 - m1_ref[...]) / s1_ref[...] * g1_ref[...] + b1_ref[...], 0.0)
    h2 = jnp.dot(r1, W2t_ref[...], preferred_element_type=jnp.float32)
    r2 = jnp.maximum(
        (h2 - m2_ref[...]) / s2_ref[...] * g2_ref[...] + b2_ref[...], 0.0)
    blk = x.shape[0]

    def dot(a, b):
        return jnp.dot(a, b, preferred_element_type=f32)

    def hdot(a1, a2, b):
        return dot(a1, b) + dot(a2, b)

    r2d = jax.lax.broadcasted_iota(jnp.int32, (L, L), 0)
    c2d = jax.lax.broadcasted_iota(jnp.int32, (L, L), 1)
    tri = ((r2d <= c2d) & ((r2d // NB) == (c2d // NB))).astype(f32)
    rg = jax.lax.broadcasted_iota(jnp.int32, (L, T), 0)
    cg = jax.lax.broadcasted_iota(jnp.int32, (L, T), 1)
    lane1 = jax.lax.broadcasted_iota(jnp.int32, (1, L), 1)
    vs = ((lane1 % NB) != 0).astype(f32)
    ve = ((lane1 % NB) != NB - 1).astype(f32)
    z1 = jnp.zeros((blk, 1), f32)

    o_ref[:, 0:P] = x[:, 0:P]
    xB8 = x[:, P:P + T]
    xB8 = jnp.where(xB8 > 1.0 - 1e-6, 1.0 - 1e-6, xB8)

    Wn8 = jnp.zeros((blk, T), f32)
    Cl8 = jnp.zeros((blk, T), f32)
    Wd8 = jnp.zeros((blk, T), f32)
    lo8 = jnp.zeros((blk, T), f32)
    sh8 = jnp.zeros((blk, T), f32)
    Vd8 = jnp.zeros((blk, T), f32)
    Vh8 = jnp.zeros((blk, T), f32)
    is31 = ((lane1 % NB) == NB - 1).astype(jnp.float32)
    for g in range(T // NG):
        s = slice(g, g + 1)
        G8 = ((rg // NB) == (cg - NG * g)).astype(f32)
        rE = jax.lax.broadcasted_iota(jnp.int32, (T, L), 0)
        cE = jax.lax.broadcasted_iota(jnp.int32, (T, L), 1)
        E = ((cE // NB) == (rE - NG * g)).astype(f32)
        Wr = jnp.exp(dot(r2, Ww_ref[g]) + bw_ref[s, :])
        Vlo = jnp.exp(dot(r2, Wvlo_ref[g]) + bvlo_ref[s, :])
        Vhi = jnp.exp(dot(r2, Wvhi_ref[g]) + bvhi_ref[s, :])
        Wr1 = _rnd(Wr); Wr2 = Wr - Wr1
        Wcs = hdot(Wr1, Wr2, tri)
        mid = ((Vlo + Vhi) * 0.5) * Wr
        mid1 = _rnd(mid); mid2 = mid - mid1
        Wng = hdot(Wr1, Wr2, G8)
        Wn8 = Wn8 + Wng
        Cl8 = Cl8 + hdot(mid1, mid2, G8)

        xbw = xB8 * Wng
        xbw1 = _rnd(xbw); xbw2 = xbw - xbw1
        xBW = hdot(xbw1, xbw2, E)
        cf = jnp.maximum((Wcs > xBW).astype(f32), is31)
        cfs = jnp.concatenate([z1, cf[:, :L - 1]], axis=1) * vs
        mf = cf - cfs
        ncf = 1.0 - cf

        Vlo1 = _rnd(Vlo); Vlo2 = Vlo - Vlo1
        Vhi1 = _rnd(Vhi); Vhi2 = Vhi - Vhi1
        Wd8 = Wd8 + hdot(mf * Wr1, mf * Wr2, G8)
        lo8 = lo8 + hdot(ncf * Wr1, ncf * Wr2, G8)
        sh8 = sh8 + hdot(ncf * mid1, ncf * mid2, G8)
        Vd8 = Vd8 + hdot(mf * Vlo1, mf * Vlo2, G8)
        Vh8 = Vh8 + hdot(mf * Vhi1, mf * Vhi2, G8)

    Wd = Wd8 / Wn8
    lo = lo8 / Wn8
    sh = sh8 / Cl8
    Vtot = Cl8 / Wn8
    Vd = Vd8 / Vtot
    Vd1 = Vh8 / Vtot
    alpha = (xB8 - lo) / Wd
    dV = Vd1 - Vd
    o_ref[:, P:P + T] = \
        (0.5 * alpha * alpha) * (dV * Wd) + alpha * (Vd * Wd) + sh
    l = Vd + alpha * dV
    o_ref[:, P + T:P + T + 1] = x[:, P + T:P + T + 1] * \
        ((l[:, 0:1] * l[:, 1:2]) * (l[:, 2:3] * l[:, 3:4])) * \
        ((l[:, 4:5] * l[:, 5:6]) * (l[:, 6:7] * l[:, 7:8]))


def _colspec(n):
    return pl.BlockSpec((1, n), lambda i: (0, 0))


def _accspec(n):
    return pl.BlockSpec((1, 1, n), lambda i: (i, 0, 0))


def kernel(x, bn0_g, bn0_b, W1, bn1_g, bn1_b, W2, bn2_g, bn2_b, W3, b3):
    B, C = x.shape
    f32 = jnp.float32
    n1 = B // BLK1
    xspec1 = pl.BlockSpec((BLK1, C), lambda i: (i, 0))
    accshape = lambda n: jax.ShapeDtypeStruct((n1, 1, n), f32)

    s, ss = pl.pallas_call(
        _stats_x_kernel,
        grid=(n1,),
        in_specs=[xspec1],
        out_specs=[_accspec(P), _accspec(P)],
        out_shape=[accshape(P), accshape(P)],
        compiler_params=pltpu.CompilerParams(
            dimension_semantics=("parallel",)),
    )(x)
    m0 = jnp.sum(s, axis=0) / B
    v0 = jnp.sum(ss, axis=0) / B - m0 * m0
    s0 = jnp.sqrt(v0 + EPS)
    g0 = bn0_g[None, :]
    b0 = bn0_b[None, :]
    W1t = W1.T

    s, ss = pl.pallas_call(
        _stats_h1_kernel,
        grid=(n1,),
        in_specs=[xspec1, _colspec(P), _colspec(P), _colspec(P), _colspec(P),
                  pl.BlockSpec((P, H), lambda i: (0, 0))],
        out_specs=[_accspec(H), _accspec(H)],
        out_shape=[accshape(H), accshape(H)],
        compiler_params=pltpu.CompilerParams(
            dimension_semantics=("parallel",)),
    )(x, m0, s0, g0, b0, W1t)
    m1 = jnp.sum(s, axis=0) / B
    v1 = jnp.sum(ss, axis=0) / B - m1 * m1
    s1 = jnp.sqrt(v1 + EPS)
    g1 = bn1_g[None, :]
    b1 = bn1_b[None, :]
    W2t = W2.T

    s, ss = pl.pallas_call(
        _stats_h2_kernel,
        grid=(n1,),
        in_specs=[xspec1, _colspec(P), _colspec(P), _colspec(P), _colspec(P),
                  pl.BlockSpec((P, H), lambda i: (0, 0)),
                  _colspec(H), _colspec(H), _colspec(H), _colspec(H),
                  pl.BlockSpec((H, H), lambda i: (0, 0))],
        out_specs=[_accspec(H), _accspec(H)],
        out_shape=[accshape(H), accshape(H)],
        compiler_params=pltpu.CompilerParams(
            dimension_semantics=("parallel",)),
    )(x, m0, s0, g0, b0, W1t, m1, s1, g1, b1, W2t)
    m2 = jnp.sum(s, axis=0) / B
    v2 = jnp.sum(ss, axis=0) / B - m2 * m2
    s2 = jnp.sqrt(v2 + EPS)
    g2 = bn2_g[None, :]
    b2 = bn2_b[None, :]

    NV = NB + 1
    W3r = W3.reshape(T, 2 * NB + 1, H)
    b3r = b3.reshape(T, 2 * NB + 1)
    ngrp = T // NG

    def pack(sl):
        w = W3r[:, sl, :].transpose(0, 2, 1)
        w = w.reshape(ngrp, NG, H, NB).transpose(0, 2, 1, 3)
        return w.reshape(ngrp, H, NG * NB)

    def packb(sl):
        bb = b3r[:, sl]
        return bb.reshape(ngrp, NG * NB)

    Wvlo = pack(slice(0, NB))
    Wvhi = pack(slice(1, NV))
    Ww = pack(slice(NV, NV + NB))
    bvlo = packb(slice(0, NB))
    bvhi = packb(slice(1, NV))
    bw = packb(slice(NV, NV + NB))

    n3 = B // BLK3
    wspec = pl.BlockSpec((ngrp, H, NG * NB), lambda i: (0, 0, 0))
    bspec = pl.BlockSpec((ngrp, NG * NB), lambda i: (0, 0))
    out = pl.pallas_call(
        _final_kernel,
        grid=(n3,),
        in_specs=[pl.BlockSpec((BLK3, C), lambda i: (i, 0)),
                  _colspec(P), _colspec(P), _colspec(P), _colspec(P),
                  pl.BlockSpec((P, H), lambda i: (0, 0)),
                  _colspec(H), _colspec(H), _colspec(H), _colspec(H),
                  pl.BlockSpec((H, H), lambda i: (0, 0)),
                  _colspec(H), _colspec(H), _colspec(H), _colspec(H),
                  wspec, wspec, wspec, bspec, bspec, bspec],
        out_specs=pl.BlockSpec((BLK3, C), lambda i: (i, 0)),
        out_shape=jax.ShapeDtypeStruct((B, C), f32),
        compiler_params=pltpu.CompilerParams(
            dimension_semantics=("parallel",)),
    )(x, m0, s0, g0, b0, W1t, m1, s1, g1, b1, W2t, m2, s2, g2, b2,
      Wvlo, Wvhi, Ww, bvlo, bvhi, bw)
    return out
